# baseline (speedup 1.0000x reference)
import functools

import jax
import jax.numpy as jnp
from jax import lax
from jax.experimental import pallas as pl
from jax.experimental.pallas import tpu as pltpu
from jax.experimental.pallas import tpu_sc as plsc

_HOP = 256
_SEG = 512
_BLK = 32768
_T = 64
_MAIN = 4088
_TAIL = 7


def _window_kernel(x_ref, xn_ref, w_ref, y0_ref, y1_ref):
    bt = _BLK // _HOP
    v3 = x_ref[...].reshape(8, bt, _HOP)
    vb = xn_ref[...].reshape(8, 1, _HOP)
    y0_ref[...] = v3 * w_ref[0, :]
    shifted = jnp.concatenate([v3[:, 1:, :], vb], axis=1)
    y1_ref[...] = shifted * w_ref[1, :]


def _tc_windowed(x, analysis_window):
    batch, num_samples = x.shape
    num_chunks = num_samples // _HOP
    bt = _BLK // _HOP
    nj = num_samples // _BLK
    w2 = analysis_window.reshape(2, _HOP)
    return pl.pallas_call(
        _window_kernel,
        grid=(batch // 8, nj),
        in_specs=[
            pl.BlockSpec((8, _BLK), lambda i, j: (i, j)),
            pl.BlockSpec((8, _HOP),
                         lambda i, j: (i, jnp.minimum((j + 1) * bt,
                                                      num_chunks - 1))),
            pl.BlockSpec((2, _HOP), lambda i, j: (0, 0)),
        ],
        out_specs=[
            pl.BlockSpec((8, bt, _HOP), lambda i, j: (i, j, 0)),
            pl.BlockSpec((8, bt, _HOP), lambda i, j: (i, j, 0)),
        ],
        out_shape=[
            jax.ShapeDtypeStruct((batch, num_chunks, _HOP), x.dtype),
            jax.ShapeDtypeStruct((batch, num_chunks, _HOP), x.dtype),
        ],
    )(x, x, w2)


def _sc_assemble(y0c, y1s, batch, num_frames):
    mesh = plsc.VectorSubcoreMesh(core_axis_name="c", subcore_axis_name="s")
    ntiles = _MAIN // _T + (1 if _MAIN % _T else 0)
    last_sz = _MAIN - (_MAIN // _T) * _T or _T
    total = batch * ntiles
    nwork = 32
    per_worker = total // nwork

    @functools.partial(
        pl.kernel,
        out_type=jax.ShapeDtypeStruct((batch, num_frames, _SEG), y0c.dtype),
        mesh=mesh,
        scratch_types=[
            pltpu.VMEM((_T, _HOP), y0c.dtype),
            pltpu.VMEM((_T, _HOP), y0c.dtype),
        ],
    )
    def assemble(y0_hbm, y1_hbm, out_hbm, v0, v1):
        wid = lax.axis_index("s") * 2 + lax.axis_index("c")

        @pl.loop(0, per_worker)
        def _(i):
            g = i * nwork + wid
            b = g // ntiles
            t = g % ntiles
            k0 = t * _T

            @pl.when(t < ntiles - 1)
            def _full():
                pltpu.sync_copy(y0_hbm.at[b, pl.ds(k0, _T), :], v0)
                pltpu.sync_copy(y1_hbm.at[b, pl.ds(k0, _T), :], v1)
                pltpu.sync_copy(v0, out_hbm.at[b, pl.ds(k0, _T), 0:_HOP])
                pltpu.sync_copy(v1, out_hbm.at[b, pl.ds(k0, _T), _HOP:_SEG])

            @pl.when(t == ntiles - 1)
            def _last():
                sz = last_sz
                pltpu.sync_copy(y0_hbm.at[b, pl.ds(k0, sz), :],
                                v0.at[pl.ds(0, sz), :])
                pltpu.sync_copy(y1_hbm.at[b, pl.ds(k0, sz), :],
                                v1.at[pl.ds(0, sz), :])
                pltpu.sync_copy(v0.at[pl.ds(0, sz), :],
                                out_hbm.at[b, pl.ds(k0, sz), 0:_HOP])
                pltpu.sync_copy(v1.at[pl.ds(0, sz), :],
                                out_hbm.at[b, pl.ds(k0, sz), _HOP:_SEG])

    return assemble(y0c, y1s)


def _tail_kernel(y0t_ref, y1t_ref, _, o_hbm, scratch, sem):
    batch = scratch.shape[0]
    scratch[:, :, 0:_HOP] = y0t_ref[:, 0:_TAIL, :]
    scratch[:, :, _HOP:_SEG] = y1t_ref[:, 0:_TAIL, :]
    cp = pltpu.make_async_copy(
        scratch, o_hbm.at[:, pl.ds(_MAIN, _TAIL), :], sem)
    cp.start()
    cp.wait()


def _tc_tail_fix(y0c, y1s, out):
    batch, num_chunks, _ = y0c.shape
    num_frames = out.shape[1]
    return pl.pallas_call(
        _tail_kernel,
        grid=(1,),
        in_specs=[
            pl.BlockSpec((batch, 8, _HOP), lambda i: (0, num_chunks // 8 - 1, 0)),
            pl.BlockSpec((batch, 8, _HOP), lambda i: (0, num_chunks // 8 - 1, 0)),
            pl.BlockSpec(memory_space=pltpu.MemorySpace.HBM),
        ],
        out_specs=pl.BlockSpec(memory_space=pltpu.MemorySpace.HBM),
        out_shape=jax.ShapeDtypeStruct(out.shape, out.dtype),
        scratch_shapes=[
            pltpu.VMEM((batch, _TAIL, _SEG), out.dtype),
            pltpu.SemaphoreType.DMA,
        ],
        input_output_aliases={2: 0},
    )(y0c, y1s, out)


def kernel(x, analysis_window):
    batch, num_samples = x.shape
    num_frames = (num_samples - _SEG) // _HOP + 1
    y0c, y1s = _tc_windowed(x, analysis_window)
    out = _sc_assemble(y0c, y1s, batch, num_frames)
    return _tc_tail_fix(y0c, y1s, out)


# speedup vs baseline: 1.4156x; 1.4156x over previous
import jax
import jax.numpy as jnp
from jax.experimental import pallas as pl
from jax.experimental.pallas import tpu as pltpu

_HOP = 256
_SEG = 512
_KT = 512
_NT = 8
_MAIN = 4088
_TAIL = 7


def _frames_kernel(x_ref, w_ref, o_hbm, tail_ref, scratch, sems):
    b = pl.program_id(0)
    nb = pl.num_programs(0)
    w0 = w_ref[0, :]
    w1 = w_ref[1, :]

    starts = [t * _KT for t in range(_NT)]
    sizes = [min(_KT, _MAIN - t * _KT) for t in range(_NT)]

    def t_copy(t, row):
        return pltpu.make_async_copy(
            scratch.at[t, pl.ds(0, sizes[t]), :],
            o_hbm.at[row, pl.ds(starts[t], sizes[t]), :],
            sems.at[t],
        )

    for t in range(_NT):
        k0, sz = starts[t], sizes[t]

        @pl.when(b >= 1)
        def _drain_prev(t=t):
            t_copy(t, b - 1).wait()

        scratch[t, 0:sz, 0:_HOP] = x_ref[0, k0:k0 + sz, :] * w0
        scratch[t, 0:sz, _HOP:_SEG] = x_ref[0, k0 + 1:k0 + sz + 1, :] * w1
        t_copy(t, b).start()

    tail_ref[0, :, 0:_HOP] = x_ref[0, _MAIN:_MAIN + _TAIL, :] * w0
    tail_ref[0, :, _HOP:_SEG] = x_ref[0, _MAIN + 1:_MAIN + _TAIL + 1, :] * w1

    @pl.when(b == nb - 1)
    def _drain_tail():
        for t in range(_NT):
            t_copy(t, b).wait()


def _tail_kernel(tail_ref, _, o_hbm, sem):
    cp = pltpu.make_async_copy(
        tail_ref, o_hbm.at[:, pl.ds(_MAIN, _TAIL), :], sem)
    cp.start()
    cp.wait()


def _tail_fix(tail, out):
    batch = out.shape[0]
    return pl.pallas_call(
        _tail_kernel,
        grid=(1,),
        in_specs=[
            pl.BlockSpec(memory_space=pltpu.MemorySpace.HBM),
            pl.BlockSpec(memory_space=pltpu.MemorySpace.HBM),
        ],
        out_specs=pl.BlockSpec(memory_space=pltpu.MemorySpace.HBM),
        out_shape=jax.ShapeDtypeStruct(out.shape, out.dtype),
        scratch_shapes=[
            pltpu.SemaphoreType.DMA,
        ],
        input_output_aliases={1: 0},
    )(tail, out)


def kernel(x, analysis_window):
    batch, num_samples = x.shape
    num_chunks = num_samples // _HOP
    num_frames = (num_samples - _SEG) // _HOP + 1

    x3 = x.reshape(batch, num_chunks, _HOP)
    w2 = analysis_window.reshape(2, _HOP)

    main, tail = pl.pallas_call(
        _frames_kernel,
        grid=(batch,),
        in_specs=[
            pl.BlockSpec((1, num_chunks, _HOP), lambda b: (b, 0, 0)),
            pl.BlockSpec((2, _HOP), lambda b: (0, 0)),
        ],
        out_specs=[
            pl.BlockSpec(memory_space=pltpu.MemorySpace.HBM),
            pl.BlockSpec((1, _TAIL, _SEG), lambda b: (b, 0, 0)),
        ],
        out_shape=[
            jax.ShapeDtypeStruct((batch, num_frames, _SEG), x.dtype),
            jax.ShapeDtypeStruct((batch, _TAIL, _SEG), x.dtype),
        ],
        scratch_shapes=[
            pltpu.VMEM((_NT, _KT, _SEG), x.dtype),
            pltpu.SemaphoreType.DMA((_NT,)),
        ],
    )(x3, w2)
    return _tail_fix(tail, main)


# speedup vs baseline: 2.0074x; 1.4180x over previous
import jax
import jax.numpy as jnp
from jax.experimental import pallas as pl

_HOP = 256
_SEG = 512
_KT = 512


def _frames_kernel(x_ref, xh_ref, w_ref, o_ref):
    v3 = x_ref[...].reshape(8, _KT, _HOP)
    vh = xh_ref[...].reshape(8, 1, _HOP)
    shifted = jnp.concatenate([v3[:, 1:, :], vh], axis=1)
    o_ref[...] = jnp.concatenate(
        [v3 * w_ref[0, :], shifted * w_ref[1, :]], axis=2)


def kernel(x, analysis_window):
    batch, num_samples = x.shape
    num_chunks = num_samples // _HOP
    num_frames = (num_samples - _SEG) // _HOP + 1
    nj = num_chunks // _KT
    w2 = analysis_window.reshape(2, _HOP)

    full = pl.pallas_call(
        _frames_kernel,
        grid=(batch // 8, nj),
        in_specs=[
            pl.BlockSpec((8, _KT * _HOP), lambda i, j: (i, j)),
            pl.BlockSpec((8, _HOP),
                         lambda i, j: (i, jnp.minimum((j + 1) * _KT,
                                                      num_chunks - 1))),
            pl.BlockSpec((2, _HOP), lambda i, j: (0, 0)),
        ],
        out_specs=pl.BlockSpec((8, _KT, _SEG), lambda i, j: (i, j, 0)),
        out_shape=jax.ShapeDtypeStruct((batch, num_chunks, _SEG), x.dtype),
    )(x, x, w2)
    return full[:, :num_frames, :]
